# baseline (speedup 1.0000x reference)
import functools

import jax
import jax.numpy as jnp
from jax import lax
from jax.experimental import pallas as pl
from jax.experimental.pallas import tpu as pltpu
from jax.experimental.pallas import tpu_sc as plsc

N = 10000
E = 320000
D = 128
H = 8
DH = 16

NC, NS = 2, 16
NW = NC * NS
NP = 10000
RPT = NP // NS
EPT = E // NW
C = 40
NCH = EPT // C
NSLOT = 5

BN = 1000
NB = N // BN

_f32 = jnp.float32


def _dense_pre_body(x_ref, w_ref, alf_ref, arf_ref, sel_ref,
                    feat_ref, el_ref, er_ref, mel_ref):
    i = pl.program_id(0)
    feat = jnp.dot(x_ref[...], w_ref[...], preferred_element_type=_f32)
    feat_ref[...] = feat
    el = jnp.dot(feat * alf_ref[...], sel_ref[...], preferred_element_type=_f32)
    er = jnp.dot(feat * arf_ref[...], sel_ref[...], preferred_element_type=_f32)
    el_ref[...] = el
    er_ref[...] = er
    bm = jnp.max(el, axis=0, keepdims=True)

    @pl.when(i == 0)
    def _():
        mel_ref[...] = bm

    @pl.when(i > 0)
    def _():
        mel_ref[...] = jnp.maximum(mel_ref[...], bm)


def _dense_pre(x, w, alf, arf, sel):
    return pl.pallas_call(
        _dense_pre_body,
        grid=(NB,),
        in_specs=[
            pl.BlockSpec((BN, D), lambda i: (i, 0)),
            pl.BlockSpec((D, D), lambda i: (0, 0)),
            pl.BlockSpec((1, D), lambda i: (0, 0)),
            pl.BlockSpec((1, D), lambda i: (0, 0)),
            pl.BlockSpec((D, H), lambda i: (0, 0)),
        ],
        out_specs=[
            pl.BlockSpec((BN, D), lambda i: (i, 0)),
            pl.BlockSpec((BN, H), lambda i: (i, 0)),
            pl.BlockSpec((BN, H), lambda i: (i, 0)),
            pl.BlockSpec((1, H), lambda i: (0, 0)),
        ],
        out_shape=[
            jax.ShapeDtypeStruct((N, D), _f32),
            jax.ShapeDtypeStruct((N, H), _f32),
            jax.ShapeDtypeStruct((N, H), _f32),
            jax.ShapeDtypeStruct((1, H), _f32),
        ],
    )(x, w, alf, arf, sel)


def _dense_mid_body(acc_ref, den_ref, brow_ref, w_ref, alf_ref, arf_ref,
                    sel_ref, selt_ref,
                    feat_ref, el_ref, er_ref, mel_ref):
    i = pl.program_id(0)
    acc = acc_ref[0] + acc_ref[1]
    den = den_ref[0] + den_ref[1]
    deninv = 1.0 / (den + 1e-9)
    dexp = jnp.dot(deninv, selt_ref[...], preferred_element_type=_f32)
    h = acc * dexp + brow_ref[...]
    h = jnp.where(h > 0, h, jnp.exp(jnp.minimum(h, 0.0)) - 1.0)
    feat = jnp.dot(h, w_ref[...], preferred_element_type=_f32)
    feat_ref[...] = feat
    el = jnp.dot(feat * alf_ref[...], sel_ref[...], preferred_element_type=_f32)
    er = jnp.dot(feat * arf_ref[...], sel_ref[...], preferred_element_type=_f32)
    el_ref[...] = el
    er_ref[...] = er
    bm = jnp.max(el, axis=0, keepdims=True)

    @pl.when(i == 0)
    def _():
        mel_ref[...] = bm

    @pl.when(i > 0)
    def _():
        mel_ref[...] = jnp.maximum(mel_ref[...], bm)


def _dense_mid(acc, den, brow, w, alf, arf, sel, selt):
    return pl.pallas_call(
        _dense_mid_body,
        grid=(NB,),
        in_specs=[
            pl.BlockSpec((NC, BN, D), lambda i: (0, i, 0)),
            pl.BlockSpec((NC, BN, H), lambda i: (0, i, 0)),
            pl.BlockSpec((1, D), lambda i: (0, 0)),
            pl.BlockSpec((D, D), lambda i: (0, 0)),
            pl.BlockSpec((1, D), lambda i: (0, 0)),
            pl.BlockSpec((1, D), lambda i: (0, 0)),
            pl.BlockSpec((D, H), lambda i: (0, 0)),
            pl.BlockSpec((H, D), lambda i: (0, 0)),
        ],
        out_specs=[
            pl.BlockSpec((BN, D), lambda i: (i, 0)),
            pl.BlockSpec((BN, H), lambda i: (i, 0)),
            pl.BlockSpec((BN, H), lambda i: (i, 0)),
            pl.BlockSpec((1, H), lambda i: (0, 0)),
        ],
        out_shape=[
            jax.ShapeDtypeStruct((N, D), _f32),
            jax.ShapeDtypeStruct((N, H), _f32),
            jax.ShapeDtypeStruct((N, H), _f32),
            jax.ShapeDtypeStruct((1, H), _f32),
        ],
    )(acc, den, brow, w, alf, arf, sel, selt)


def _dense_fin_body(acc_ref, den_ref, brow_ref, selt_ref, out_ref):
    acc = acc_ref[0] + acc_ref[1]
    den = den_ref[0] + den_ref[1]
    deninv = 1.0 / (den + 1e-9)
    dexp = jnp.dot(deninv, selt_ref[...], preferred_element_type=_f32)
    h = acc * dexp + brow_ref[...]
    out_ref[...] = jnp.where(h > 0, h, jnp.exp(jnp.minimum(h, 0.0)) - 1.0)


def _dense_fin(acc, den, brow, selt):
    return pl.pallas_call(
        _dense_fin_body,
        grid=(NB,),
        in_specs=[
            pl.BlockSpec((NC, BN, D), lambda i: (0, i, 0)),
            pl.BlockSpec((NC, BN, H), lambda i: (0, i, 0)),
            pl.BlockSpec((1, D), lambda i: (0, 0)),
            pl.BlockSpec((H, D), lambda i: (0, 0)),
        ],
        out_specs=pl.BlockSpec((BN, D), lambda i: (i, 0)),
        out_shape=jax.ShapeDtypeStruct((N, D), _f32),
    )(acc, den, brow, selt)


def _sc_edge_body(ed_h, el_h, er_h, feat_h, mel_h, z128_h, z8_h,
                  acc_o, den_o, *scratch):
    slots = [tuple(scratch[i * 5:(i + 1) * 5]) for i in range(NSLOT)]
    melv = scratch[5 * NSLOT]
    acc_sh = scratch[5 * NSLOT + 1]
    den_sh = scratch[5 * NSLOT + 2]
    sems = scratch[5 * NSLOT + 3:]
    for i in range(NSLOT):
        slots[i] = slots[i] + tuple(sems[i * 3:(i + 1) * 3])

    cid = lax.axis_index("c")
    sid = lax.axis_index("s")
    wid = cid * NS + sid
    row0 = sid * RPT

    pltpu.sync_copy(mel_h, melv)
    pltpu.sync_copy(z128_h.at[pl.ds(row0, RPT)], acc_sh.at[pl.ds(row0, RPT)])
    pltpu.sync_copy(z8_h.at[pl.ds(row0, RPT)], den_sh.at[pl.ds(row0, RPT)])
    plsc.subcore_barrier()

    cbase = wid * NCH

    def issue_idx(c, b):
        pltpu.async_copy(ed_h.at[cbase + c], b[0], b[5])

    def wait_idx(b):
        pltpu.make_async_copy(ed_h.at[0], b[0], b[5]).wait()

    def issue_gath(b):
        pltpu.async_copy(el_h.at[b[0].at[0]], b[1], b[6])
        pltpu.async_copy(er_h.at[b[0].at[1]], b[2], b[6])
        pltpu.async_copy(feat_h.at[b[0].at[0]], b[3], b[6])

    def wait_gath(b):
        pltpu.make_async_copy(el_h.at[b[0].at[0]], b[1], b[6]).wait()
        pltpu.make_async_copy(er_h.at[b[0].at[1]], b[2], b[6]).wait()
        pltpu.make_async_copy(feat_h.at[b[0].at[0]], b[3], b[6]).wait()

    def issue_scat(b):
        pltpu.async_copy(b[4], den_sh.at[b[0].at[1]], b[7], add=True)
        pltpu.async_copy(b[3], acc_sh.at[b[0].at[1]], b[7], add=True)

    def wait_scat(b):
        pltpu.make_async_copy(b[4], den_sh.at[b[0].at[1]], b[7]).wait()
        pltpu.make_async_copy(b[3], acc_sh.at[b[0].at[1]], b[7]).wait()

    def compute(b):
        ebuf, elg, erg, featg, pch = b[0], b[1], b[2], b[3], b[4]

        @plsc.parallel_loop(0, C // 2, 1, unroll=2)
        def pair_body(i):
            it = lax.iota(jnp.int32, 16)
            half = lax.shift_right_logical(it, 3)
            col8 = jnp.bitwise_and(it, 7)
            e0 = 2 * i
            row2 = e0 + half
            el2 = plsc.load_gather(elg, [row2, col8])
            er2 = plsc.load_gather(erg, [row2, col8])
            ewi = plsc.load_gather(ebuf, [jnp.full((16,), 2, jnp.int32), row2])
            ew2 = plsc.bitcast(ewi, _f32)
            mel2 = melv[...]
            t = er2 + mel2
            b2 = jnp.maximum(t, 0.2 * t)
            s = el2 + er2
            lr = jnp.maximum(s, 0.2 * s)
            p = jnp.exp(lr - b2)
            q = p * ew2
            plsc.store_scatter(pch, [row2, col8], p)
            for sub in range(2):
                e = e0 + sub
                for hh in range(H):
                    qs = jnp.take(q, jnp.full((16,), sub * H + hh, jnp.int32),
                                  mode="fill")
                    featg[e, pl.ds(hh * DH, DH)] = featg[e, pl.ds(hh * DH, DH)] * qs

    issue_idx(0, slots[0])
    issue_idx(1, slots[1])
    wait_idx(slots[0])
    issue_gath(slots[0])

    for k in range(NSLOT):
        if k >= 2:
            wait_scat(slots[(k + 3) % NSLOT])
        issue_idx(k + 2, slots[(k + 2) % NSLOT])
        wait_idx(slots[(k + 1) % NSLOT])
        issue_gath(slots[(k + 1) % NSLOT])
        wait_gath(slots[k % NSLOT])
        compute(slots[k % NSLOT])
        issue_scat(slots[k % NSLOT])

    NG = NCH // NSLOT

    def group_body(g, carry):
        k0 = NSLOT * g
        for m in range(NSLOT):
            k = k0 + m
            wait_scat(slots[(m + 3) % NSLOT])
            issue_idx(k + 2, slots[(m + 2) % NSLOT])
            wait_idx(slots[(m + 1) % NSLOT])
            issue_gath(slots[(m + 1) % NSLOT])
            wait_gath(slots[m])
            compute(slots[m])
            issue_scat(slots[m])
        return carry

    lax.fori_loop(1, NG - 1, group_body, 0)

    for m in range(NSLOT):
        k = NCH - NSLOT + m
        wait_scat(slots[(m + 3) % NSLOT])
        if k + 2 < NCH:
            issue_idx(k + 2, slots[(m + 2) % NSLOT])
        if k + 1 < NCH:
            wait_idx(slots[(m + 1) % NSLOT])
            issue_gath(slots[(m + 1) % NSLOT])
        wait_gath(slots[m])
        compute(slots[m])
        issue_scat(slots[m])
    wait_scat(slots[(NCH - 2) % NSLOT])
    wait_scat(slots[(NCH - 1) % NSLOT])

    plsc.subcore_barrier()
    pltpu.sync_copy(acc_sh.at[pl.ds(row0, RPT)], acc_o.at[cid, pl.ds(row0, RPT)])
    pltpu.sync_copy(den_sh.at[pl.ds(row0, RPT)], den_o.at[cid, pl.ds(row0, RPT)])


@functools.cache
def _get_sc_edge_pass():
  slot_scratch = []
  for _ in range(NSLOT):
    slot_scratch += [
        pltpu.VMEM((3, C), jnp.int32),
        pltpu.VMEM((C, H), _f32),
        pltpu.VMEM((C, H), _f32),
        pltpu.VMEM((C, D), _f32),
        pltpu.VMEM((C, H), _f32),
    ]
  return functools.partial(
    pl.kernel,
    out_type=(jax.ShapeDtypeStruct((NC, NP, D), _f32),
              jax.ShapeDtypeStruct((NC, NP, H), _f32)),
    mesh=plsc.VectorSubcoreMesh(core_axis_name="c", subcore_axis_name="s",
                                num_cores=NC, num_subcores=NS),
    compiler_params=pltpu.CompilerParams(use_tc_tiling_on_sc=False,
                                         needs_layout_passes=False),
    scratch_types=(
        slot_scratch
        + [pltpu.VMEM((16,), _f32)]
        + [pltpu.VMEM_SHARED((NP, D), _f32),
           pltpu.VMEM_SHARED((NP, H), _f32)]
        + [pltpu.SemaphoreType.DMA] * (3 * NSLOT)
    ),
  )(_sc_edge_body)


def kernel(in_feat, edge_index, edge_weights, W1, attn_l1, attn_r1, b1,
           W2, attn_l2, attn_r2, b2):
    src = edge_index[0]
    dst = edge_index[1]
    ewbits = lax.bitcast_convert_type(edge_weights, jnp.int32)
    edata = jnp.stack([src.reshape(E // C, C), dst.reshape(E // C, C),
                       ewbits.reshape(E // C, C)], axis=1)
    sel = jnp.repeat(jnp.eye(H, dtype=_f32), DH, axis=0)
    selt = sel.T
    alf1 = attn_l1.reshape(1, D)
    arf1 = attn_r1.reshape(1, D)
    alf2 = attn_l2.reshape(1, D)
    arf2 = attn_r2.reshape(1, D)
    z128 = jnp.zeros((NP, D), _f32)
    z8 = jnp.zeros((NP, H), _f32)

    feat1, el1, er1, mel1 = _dense_pre(in_feat, W1, alf1, arf1, sel)
    mel16_1 = jnp.concatenate([mel1, mel1], axis=1).reshape(16)
    sc_pass = _get_sc_edge_pass()
    acc1, den1 = sc_pass(edata, el1, er1, feat1, mel16_1, z128, z8)
    feat2, el2, er2, mel2 = _dense_mid(acc1, den1, b1.reshape(1, D), W2,
                                       alf2, arf2, sel, selt)
    mel16_2 = jnp.concatenate([mel2, mel2], axis=1).reshape(16)
    acc2, den2 = sc_pass(edata, el2, er2, feat2, mel16_2, z128, z8)
    out = _dense_fin(acc2, den2, b2.reshape(1, D), selt)
    return out


# speedup vs baseline: 1.0937x; 1.0937x over previous
import functools

import jax
import jax.numpy as jnp
from jax import lax
from jax.experimental import pallas as pl
from jax.experimental.pallas import tpu as pltpu
from jax.experimental.pallas import tpu_sc as plsc

N = 10000
E = 320000
D = 128
H = 8
DH = 16

NC, NS = 2, 16
NW = NC * NS
NP = 10000
RPT = NP // NS
EPT = E // NW
C = 80
NCH = EPT // C

BN = 1000
NB = N // BN

_f32 = jnp.float32


def _dense_pre_body(x_ref, w_ref, alf_ref, arf_ref, sel_ref,
                    feat_ref, el_ref, er_ref, mel_ref):
    i = pl.program_id(0)
    feat = jnp.dot(x_ref[...], w_ref[...], preferred_element_type=_f32)
    feat_ref[...] = feat
    el = jnp.dot(feat * alf_ref[...], sel_ref[...], preferred_element_type=_f32)
    er = jnp.dot(feat * arf_ref[...], sel_ref[...], preferred_element_type=_f32)
    el_ref[...] = el
    er_ref[...] = er
    bm = jnp.max(el, axis=0, keepdims=True)

    @pl.when(i == 0)
    def _():
        mel_ref[...] = bm

    @pl.when(i > 0)
    def _():
        mel_ref[...] = jnp.maximum(mel_ref[...], bm)


def _dense_pre(x, w, alf, arf, sel):
    return pl.pallas_call(
        _dense_pre_body,
        grid=(NB,),
        in_specs=[
            pl.BlockSpec((BN, D), lambda i: (i, 0)),
            pl.BlockSpec((D, D), lambda i: (0, 0)),
            pl.BlockSpec((1, D), lambda i: (0, 0)),
            pl.BlockSpec((1, D), lambda i: (0, 0)),
            pl.BlockSpec((D, H), lambda i: (0, 0)),
        ],
        out_specs=[
            pl.BlockSpec((BN, D), lambda i: (i, 0)),
            pl.BlockSpec((BN, H), lambda i: (i, 0)),
            pl.BlockSpec((BN, H), lambda i: (i, 0)),
            pl.BlockSpec((1, H), lambda i: (0, 0)),
        ],
        out_shape=[
            jax.ShapeDtypeStruct((N, D), _f32),
            jax.ShapeDtypeStruct((N, H), _f32),
            jax.ShapeDtypeStruct((N, H), _f32),
            jax.ShapeDtypeStruct((1, H), _f32),
        ],
    )(x, w, alf, arf, sel)


def _dstt_body(er_ref, mel_ref, dt_ref):
    er = er_ref[...]
    t = er + mel_ref[...]
    b = jnp.maximum(t, 0.2 * t)
    dt_ref[...] = jnp.concatenate([er, b], axis=1)


def _mk_dstt(er, mel):
    return pl.pallas_call(
        _dstt_body,
        grid=(NB,),
        in_specs=[
            pl.BlockSpec((BN, H), lambda i: (i, 0)),
            pl.BlockSpec((1, H), lambda i: (0, 0)),
        ],
        out_specs=pl.BlockSpec((BN, 2 * H), lambda i: (i, 0)),
        out_shape=jax.ShapeDtypeStruct((N, 2 * H), _f32),
    )(er, mel)


def _dense_mid_body(acc_ref, den_ref, brow_ref, w_ref, alf_ref, arf_ref,
                    sel_ref, selt_ref,
                    feat_ref, el_ref, er_ref, mel_ref):
    i = pl.program_id(0)
    acc = acc_ref[0] + acc_ref[1]
    den = den_ref[0] + den_ref[1]
    deninv = 1.0 / (den + 1e-9)
    dexp = jnp.dot(deninv, selt_ref[...], preferred_element_type=_f32)
    h = acc * dexp + brow_ref[...]
    h = jnp.where(h > 0, h, jnp.exp(jnp.minimum(h, 0.0)) - 1.0)
    feat = jnp.dot(h, w_ref[...], preferred_element_type=_f32)
    feat_ref[...] = feat
    el = jnp.dot(feat * alf_ref[...], sel_ref[...], preferred_element_type=_f32)
    er = jnp.dot(feat * arf_ref[...], sel_ref[...], preferred_element_type=_f32)
    el_ref[...] = el
    er_ref[...] = er
    bm = jnp.max(el, axis=0, keepdims=True)

    @pl.when(i == 0)
    def _():
        mel_ref[...] = bm

    @pl.when(i > 0)
    def _():
        mel_ref[...] = jnp.maximum(mel_ref[...], bm)


def _dense_mid(acc, den, brow, w, alf, arf, sel, selt):
    return pl.pallas_call(
        _dense_mid_body,
        grid=(NB,),
        in_specs=[
            pl.BlockSpec((NC, BN, D), lambda i: (0, i, 0)),
            pl.BlockSpec((NC, BN, H), lambda i: (0, i, 0)),
            pl.BlockSpec((1, D), lambda i: (0, 0)),
            pl.BlockSpec((D, D), lambda i: (0, 0)),
            pl.BlockSpec((1, D), lambda i: (0, 0)),
            pl.BlockSpec((1, D), lambda i: (0, 0)),
            pl.BlockSpec((D, H), lambda i: (0, 0)),
            pl.BlockSpec((H, D), lambda i: (0, 0)),
        ],
        out_specs=[
            pl.BlockSpec((BN, D), lambda i: (i, 0)),
            pl.BlockSpec((BN, H), lambda i: (i, 0)),
            pl.BlockSpec((BN, H), lambda i: (i, 0)),
            pl.BlockSpec((1, H), lambda i: (0, 0)),
        ],
        out_shape=[
            jax.ShapeDtypeStruct((N, D), _f32),
            jax.ShapeDtypeStruct((N, H), _f32),
            jax.ShapeDtypeStruct((N, H), _f32),
            jax.ShapeDtypeStruct((1, H), _f32),
        ],
    )(acc, den, brow, w, alf, arf, sel, selt)


def _dense_fin_body(acc_ref, den_ref, brow_ref, selt_ref, out_ref):
    acc = acc_ref[0] + acc_ref[1]
    den = den_ref[0] + den_ref[1]
    deninv = 1.0 / (den + 1e-9)
    dexp = jnp.dot(deninv, selt_ref[...], preferred_element_type=_f32)
    h = acc * dexp + brow_ref[...]
    out_ref[...] = jnp.where(h > 0, h, jnp.exp(jnp.minimum(h, 0.0)) - 1.0)


def _dense_fin(acc, den, brow, selt):
    return pl.pallas_call(
        _dense_fin_body,
        grid=(NB,),
        in_specs=[
            pl.BlockSpec((NC, BN, D), lambda i: (0, i, 0)),
            pl.BlockSpec((NC, BN, H), lambda i: (0, i, 0)),
            pl.BlockSpec((1, D), lambda i: (0, 0)),
            pl.BlockSpec((H, D), lambda i: (0, 0)),
        ],
        out_specs=pl.BlockSpec((BN, D), lambda i: (i, 0)),
        out_shape=jax.ShapeDtypeStruct((N, D), _f32),
    )(acc, den, brow, selt)


def _sc_edge_body(src_h, dst_h, ew_h, el_h, dt_h, feat_h, z128_h, z8_h,
                  acc_o, den_o,
                  sidx0, didx0, ewv0, elg0, dtg0, featg0, pch0,
                  sidx1, didx1, ewv1, elg1, dtg1, featg1, pch1,
                  sidx2, didx2, ewv2, elg2, dtg2, featg2, pch2,
                  acc_sh, den_sh,
                  semi0, semg0, sems0, semi1, semg1, sems1, semi2, semg2, sems2):
    cid = lax.axis_index("c")
    sid = lax.axis_index("s")
    wid = cid * NS + sid
    row0 = sid * RPT

    pltpu.sync_copy(z128_h.at[pl.ds(row0, RPT)], acc_sh.at[pl.ds(row0, RPT)])
    pltpu.sync_copy(z8_h.at[pl.ds(row0, RPT)], den_sh.at[pl.ds(row0, RPT)])
    plsc.subcore_barrier()

    ebase = wid * EPT
    B0 = (sidx0, didx0, ewv0, elg0, dtg0, featg0, pch0, semi0, semg0, sems0)
    B1 = (sidx1, didx1, ewv1, elg1, dtg1, featg1, pch1, semi1, semg1, sems1)
    B2 = (sidx2, didx2, ewv2, elg2, dtg2, featg2, pch2, semi2, semg2, sems2)

    def issue_idx(c, b):
        base = ebase + c * C
        pltpu.async_copy(src_h.at[pl.ds(base, C)], b[0], b[7])
        pltpu.async_copy(dst_h.at[pl.ds(base, C)], b[1], b[7])
        pltpu.async_copy(ew_h.at[pl.ds(base, C)], b[2], b[7])

    def wait_idx(b):
        pltpu.make_async_copy(src_h.at[pl.ds(0, C)], b[0], b[7]).wait()
        pltpu.make_async_copy(dst_h.at[pl.ds(0, C)], b[1], b[7]).wait()
        pltpu.make_async_copy(ew_h.at[pl.ds(0, C)], b[2], b[7]).wait()

    def issue_gath(b):
        pltpu.async_copy(el_h.at[b[0]], b[3], b[8])
        pltpu.async_copy(dt_h.at[b[1]], b[4], b[8])
        pltpu.async_copy(feat_h.at[b[0]], b[5], b[8])

    def wait_gath(b):
        pltpu.make_async_copy(el_h.at[b[0]], b[3], b[8]).wait()
        pltpu.make_async_copy(dt_h.at[b[1]], b[4], b[8]).wait()
        pltpu.make_async_copy(feat_h.at[b[0]], b[5], b[8]).wait()

    def issue_scat(b):
        pltpu.async_copy(b[6], den_sh.at[b[1]], b[9], add=True)
        pltpu.async_copy(b[5], acc_sh.at[b[1]], b[9], add=True)

    def wait_scat(b):
        pltpu.make_async_copy(b[6], den_sh.at[b[1]], b[9]).wait()
        pltpu.make_async_copy(b[5], acc_sh.at[b[1]], b[9]).wait()

    def compute(b):
        elg, dtg, featg, pch, ewv = b[3], b[4], b[5], b[6], b[2]

        @plsc.parallel_loop(0, C // 2, 1, unroll=2)
        def pair_body(i):
            it = lax.iota(jnp.int32, 16)
            half = lax.shift_right_logical(it, 3)
            col8 = jnp.bitwise_and(it, 7)
            e0 = 2 * i
            row2 = e0 + half
            el2 = plsc.load_gather(elg, [row2, col8])
            er2 = plsc.load_gather(dtg, [row2, col8])
            b2 = plsc.load_gather(dtg, [row2, col8 + 8])
            ew2 = plsc.load_gather(ewv, [row2])
            s = el2 + er2
            lr = jnp.maximum(s, 0.2 * s)
            p = jnp.exp(lr - b2)
            q = p * ew2
            plsc.store_scatter(pch, [row2, col8], p)
            for sub in range(2):
                e = e0 + sub
                for hh in range(H):
                    qs = jnp.take(q, jnp.full((16,), sub * H + hh, jnp.int32),
                                  mode="fill")
                    featg[e, pl.ds(hh * DH, DH)] = featg[e, pl.ds(hh * DH, DH)] * qs

    NJ = (NCH - 2) // 3
    issue_idx(0, B0)
    wait_idx(B0)
    issue_gath(B0)

    def pipe_body(j, carry):
        a = 3 * j
        issue_idx(a + 1, B1)
        wait_idx(B1)
        issue_gath(B1)

        @pl.when(j > 0)
        def _():
            wait_scat(B2)

        wait_gath(B0)
        compute(B0)
        issue_scat(B0)
        issue_idx(a + 2, B2)
        wait_idx(B2)
        issue_gath(B2)
        wait_gath(B1)
        compute(B1)
        issue_scat(B1)

        @pl.when(j < NJ - 1)
        def _():
            issue_idx(a + 3, B0)

        wait_scat(B0)

        @pl.when(j < NJ - 1)
        def _():
            wait_idx(B0)
            issue_gath(B0)

        wait_gath(B2)
        compute(B2)
        issue_scat(B2)
        wait_scat(B1)
        return carry

    lax.fori_loop(0, NJ, pipe_body, 0)

    for k, b in zip(range(3 * NJ, NCH), (B0, B1)):
        issue_idx(k, b)
        wait_idx(b)
        issue_gath(b)
        wait_gath(b)
        compute(b)
        pltpu.sync_copy(b[6], den_sh.at[b[1]], add=True)
        pltpu.sync_copy(b[5], acc_sh.at[b[1]], add=True)
    wait_scat(B2)

    plsc.subcore_barrier()
    pltpu.sync_copy(acc_sh.at[pl.ds(row0, RPT)], acc_o.at[cid, pl.ds(row0, RPT)])
    pltpu.sync_copy(den_sh.at[pl.ds(row0, RPT)], den_o.at[cid, pl.ds(row0, RPT)])


@functools.cache
def _get_sc_edge_pass():
  return functools.partial(
    pl.kernel,
    out_type=(jax.ShapeDtypeStruct((NC, NP, D), _f32),
              jax.ShapeDtypeStruct((NC, NP, H), _f32)),
    mesh=plsc.VectorSubcoreMesh(core_axis_name="c", subcore_axis_name="s",
                                num_cores=NC, num_subcores=NS),
    compiler_params=pltpu.CompilerParams(use_tc_tiling_on_sc=False, needs_layout_passes=False),
    scratch_types=(
        [pltpu.VMEM((C,), jnp.int32),
         pltpu.VMEM((C,), jnp.int32),
         pltpu.VMEM((C,), _f32),
         pltpu.VMEM((C, H), _f32),
         pltpu.VMEM((C, 2 * H), _f32),
         pltpu.VMEM((C, D), _f32),
         pltpu.VMEM((C, H), _f32)] * 3
        + [pltpu.VMEM_SHARED((NP, D), _f32),
           pltpu.VMEM_SHARED((NP, H), _f32)]
        + [pltpu.SemaphoreType.DMA] * 9
    ),
  )(_sc_edge_body)


def kernel(in_feat, edge_index, edge_weights, W1, attn_l1, attn_r1, b1,
           W2, attn_l2, attn_r2, b2):
    src = edge_index[0]
    dst = edge_index[1]
    sel = jnp.repeat(jnp.eye(H, dtype=_f32), DH, axis=0)
    selt = sel.T
    alf1 = attn_l1.reshape(1, D)
    arf1 = attn_r1.reshape(1, D)
    alf2 = attn_l2.reshape(1, D)
    arf2 = attn_r2.reshape(1, D)
    z128 = jnp.zeros((NP, D), _f32)
    z8 = jnp.zeros((NP, H), _f32)

    feat1, el1, er1, mel1 = _dense_pre(in_feat, W1, alf1, arf1, sel)
    dt1 = _mk_dstt(er1, mel1)
    sc_pass = _get_sc_edge_pass()
    acc1, den1 = sc_pass(src, dst, edge_weights, el1, dt1, feat1, z128, z8)
    feat2, el2, er2, mel2 = _dense_mid(acc1, den1, b1.reshape(1, D), W2,
                                       alf2, arf2, sel, selt)
    dt2 = _mk_dstt(er2, mel2)
    acc2, den2 = sc_pass(src, dst, edge_weights, el2, dt2, feat2, z128, z8)
    out = _dense_fin(acc2, den2, b2.reshape(1, D), selt)
    return out
